# Initial kernel scaffold; baseline (speedup 1.0000x reference)
import jax
import jax.numpy as jnp
from jax.experimental import pallas as pl


def kernel(feat, edge_index, W):
    raise NotImplementedError("write your pallas kernel here")


# speedup vs baseline: 1.2268x; 1.2268x over previous
import functools

import jax
import jax.numpy as jnp
from jax import lax
from jax.experimental import pallas as pl
from jax.experimental.pallas import tpu as pltpu
from jax.experimental.pallas import tpu_sc as plsc

N_N = 10000
N_E = 160000
D = 256

NC = 2
NS = 16
NW = NC * NS
L = 16

NP = 10240
NODES_W = NP // NW

_i32 = jnp.int32


def _mesh():
    return plsc.VectorSubcoreMesh(core_axis_name="c", subcore_axis_name="s")


def _wid():
    return lax.axis_index("s") * NC + lax.axis_index("c")


def _mm_body(x_ref, w_ref, o_ref):
    o_ref[...] = jnp.dot(x_ref[...], w_ref[...],
                         preferred_element_type=jnp.float32)


def _matmul(feat, W):
    return pl.pallas_call(
        _mm_body,
        out_shape=jax.ShapeDtypeStruct((N_N, D), jnp.float32),
    )(feat, W)


G_CH = 128
G_NCH = N_E // G_CH
G_IT = (G_NCH + NW - 1) // NW


def _sc_gather(table, idx):
    @functools.partial(
        pl.kernel, mesh=_mesh(),
        compiler_params=pltpu.CompilerParams(needs_layout_passes=False),
        out_type=jax.ShapeDtypeStruct((N_E, D), jnp.float32),
        scratch_types=[
            pltpu.VMEM((G_CH,), jnp.int32),
            pltpu.VMEM((G_CH, D), jnp.float32),
            pltpu.SemaphoreType.DMA,
        ],
    )
    def gk(table_hbm, idx_hbm, out_hbm, idx_v, rows_v, sem):
        w = _wid()

        def body(c, _):
            cid = c * _i32(NW) + w

            @pl.when(cid < _i32(G_NCH))
            def _():
                off = cid * _i32(G_CH)
                pltpu.sync_copy(idx_hbm.at[pl.ds(off, G_CH)], idx_v)
                pltpu.async_copy(table_hbm.at[idx_v], rows_v, sem).wait()
                pltpu.sync_copy(rows_v, out_hbm.at[pl.ds(off, G_CH)])
            return 0

        lax.fori_loop(_i32(0), _i32(G_IT), body, 0)

    return gk(table, idx)


E_BLK = 128
E_ROWS = N_E // E_BLK


def _dot_body(a_ref, b_ref, o_ref):
    o_ref[...] = jnp.sum(a_ref[...] * b_ref[...], axis=-1,
                         keepdims=True).reshape(1, 1, E_BLK)


def _edge_dots(fs3, fd3):
    z = _i32(0)
    e3 = pl.pallas_call(
        _dot_body,
        grid=(E_ROWS,),
        in_specs=[
            pl.BlockSpec((1, E_BLK, D), lambda i: (i, _i32(0), _i32(0))),
            pl.BlockSpec((1, E_BLK, D), lambda i: (i, _i32(0), _i32(0))),
        ],
        out_specs=pl.BlockSpec((1, 1, E_BLK), lambda i: (i, _i32(0), _i32(0))),
        out_shape=jax.ShapeDtypeStruct((E_ROWS, 1, E_BLK), jnp.float32),
    )(fs3, fd3)
    return e3.reshape(N_E)


S_CH = 640
S_NCH = N_E // S_CH
NEG = -3.0e38


def _sc_maxdenom(e, dst):
    @functools.partial(
        pl.kernel, mesh=_mesh(),
        compiler_params=pltpu.CompilerParams(needs_layout_passes=False),
        out_type=(jax.ShapeDtypeStruct((NP,), jnp.float32),
                  jax.ShapeDtypeStruct((NP,), jnp.float32)),
        scratch_types=[
            pltpu.VMEM((S_CH,), jnp.float32),
            pltpu.VMEM((S_CH,), jnp.int32),
            pltpu.VMEM((NODES_W, L), jnp.float32),
            pltpu.VMEM((NODES_W, L), jnp.float32),
            pltpu.VMEM((NODES_W,), jnp.float32),
            pltpu.VMEM((NODES_W,), jnp.float32),
        ],
    )
    def mk(e_hbm, dst_hbm, emax_hbm, den_hbm,
           e_v, d_v, mx_t, sm_t, mx_r, sm_r):
        w = _wid()
        lo = w * _i32(NODES_W)
        lane = lax.iota(jnp.int32, L)

        def init_row(r, _):
            mx_t[r, :] = jnp.full((L,), NEG, jnp.float32)
            sm_t[r, :] = jnp.zeros((L,), jnp.float32)
            return 0

        lax.fori_loop(_i32(0), _i32(NODES_W), init_row, 0)

        def load_chunk(c):
            off = c * _i32(S_CH)
            pltpu.sync_copy(e_hbm.at[pl.ds(off, S_CH)], e_v)
            pltpu.sync_copy(dst_hbm.at[pl.ds(off, S_CH)], d_v)

        def scan1(c, _):
            load_chunk(c)

            def grp(j, _):
                ev = e_v[pl.ds(j * _i32(L), L)]
                dv = d_v[pl.ds(j * _i32(L), L)]
                m = (dv >= lo) & (dv < lo + _i32(NODES_W))
                loc = jnp.where(m, dv - lo, 0)
                cur = plsc.load_gather(mx_t, [loc, lane], mask=m)
                plsc.store_scatter(mx_t, [loc, lane],
                                   jnp.maximum(cur, ev), mask=m)
                return 0

            lax.fori_loop(_i32(0), _i32(S_CH // L), grp, 0)
            return 0

        lax.fori_loop(_i32(0), _i32(S_NCH), scan1, 0)

        def redmax(rg, _):
            rows = lax.iota(jnp.int32, L) + rg * _i32(L)
            acc = jnp.full((L,), NEG, jnp.float32)

            def col(c, a):
                cc = jnp.full((L,), 0, jnp.int32) + c
                return jnp.maximum(a, plsc.load_gather(mx_t, [rows, cc]))

            acc = lax.fori_loop(_i32(0), _i32(L), col, acc)
            mx_r[pl.ds(rg * _i32(L), L)] = acc
            return 0

        lax.fori_loop(_i32(0), _i32(NODES_W // L), redmax, 0)

        def scan2(c, _):
            load_chunk(c)

            def grp(j, _):
                ev = e_v[pl.ds(j * _i32(L), L)]
                dv = d_v[pl.ds(j * _i32(L), L)]
                m = (dv >= lo) & (dv < lo + _i32(NODES_W))
                loc = jnp.where(m, dv - lo, 0)
                mv = plsc.load_gather(mx_r, [loc], mask=m)
                ex = jnp.where(m, jnp.exp(ev - mv), jnp.float32(0.0))
                plsc.addupdate_scatter(sm_t, [loc, lane], ex, mask=m)
                return 0

            lax.fori_loop(_i32(0), _i32(S_CH // L), grp, 0)
            return 0

        lax.fori_loop(_i32(0), _i32(S_NCH), scan2, 0)

        def redsum(rg, _):
            rows = lax.iota(jnp.int32, L) + rg * _i32(L)
            acc = jnp.zeros((L,), jnp.float32)

            def col(c, a):
                cc = jnp.full((L,), 0, jnp.int32) + c
                return a + plsc.load_gather(sm_t, [rows, cc])

            acc = lax.fori_loop(_i32(0), _i32(L), col, acc)
            sm_r[pl.ds(rg * _i32(L), L)] = acc
            return 0

        lax.fori_loop(_i32(0), _i32(NODES_W // L), redsum, 0)

        pltpu.sync_copy(mx_r, emax_hbm.at[pl.ds(lo, NODES_W)])
        pltpu.sync_copy(sm_r, den_hbm.at[pl.ds(lo, NODES_W)])

    return mk(e, dst)


W_CH = 128
W_NCH = N_E // W_CH
W_IT = (W_NCH + NW - 1) // NW


def _sc_weights(e, dst, emax, den):
    @functools.partial(
        pl.kernel, mesh=_mesh(),
        compiler_params=pltpu.CompilerParams(needs_layout_passes=False),
        out_type=jax.ShapeDtypeStruct((N_E,), jnp.float32),
        scratch_types=[
            pltpu.VMEM((W_CH,), jnp.float32),
            pltpu.VMEM((W_CH,), jnp.int32),
            pltpu.VMEM((W_CH,), jnp.float32),
            pltpu.VMEM((W_CH,), jnp.float32),
            pltpu.VMEM((W_CH,), jnp.float32),
            pltpu.SemaphoreType.DMA,
        ],
    )
    def wk(e_hbm, dst_hbm, emax_hbm, den_hbm, w_hbm,
           e_v, d_v, m_v, s_v, w_v, sem):
        w = _wid()

        def body(c, _):
            cid = c * _i32(NW) + w

            @pl.when(cid < _i32(W_NCH))
            def _():
                off = cid * _i32(W_CH)
                pltpu.sync_copy(e_hbm.at[pl.ds(off, W_CH)], e_v)
                pltpu.sync_copy(dst_hbm.at[pl.ds(off, W_CH)], d_v)
                pltpu.async_copy(emax_hbm.at[d_v], m_v, sem).wait()
                pltpu.async_copy(den_hbm.at[d_v], s_v, sem).wait()

                def grp(j, _):
                    sl = pl.ds(j * _i32(L), L)
                    w_v[sl] = jnp.exp(e_v[sl] - m_v[sl]) / s_v[sl]
                    return 0

                lax.fori_loop(_i32(0), _i32(W_CH // L), grp, 0)
                pltpu.sync_copy(w_v, w_hbm.at[pl.ds(off, W_CH)])
            return 0

        lax.fori_loop(_i32(0), _i32(W_IT), body, 0)

    return wk(e, dst, emax, den)


def _attn_body(a_ref, w_ref, o_ref):
    o_ref[...] = a_ref[...] * w_ref[...].reshape(1, E_BLK, 1)


def _attn_scale(fs3, w2):
    return pl.pallas_call(
        _attn_body,
        grid=(E_ROWS,),
        in_specs=[
            pl.BlockSpec((1, E_BLK, D), lambda i: (i, _i32(0), _i32(0))),
            pl.BlockSpec((1, 1, E_BLK), lambda i: (i, _i32(0), _i32(0))),
        ],
        out_specs=pl.BlockSpec((1, E_BLK, D), lambda i: (i, _i32(0), _i32(0))),
        out_shape=jax.ShapeDtypeStruct((E_ROWS, E_BLK, D), jnp.float32),
    )(fs3, w2)


DH = D // NC
R_CH = 128
R_NCH = N_E // R_CH
R_IT = (R_NCH + NS - 1) // NS
STRIPE = NP // NS


def _sc_scatter_rows(attn, dst):
    @functools.partial(
        pl.kernel, mesh=_mesh(),
        compiler_params=pltpu.CompilerParams(needs_layout_passes=False),
        out_type=jax.ShapeDtypeStruct((NP, D), jnp.float32),
        scratch_types=[
            pltpu.VMEM((R_CH,), jnp.int32),
            pltpu.VMEM((R_CH, DH), jnp.float32),
            pltpu.VMEM_SHARED((NP, DH), jnp.float32),
        ],
    )
    def rk(attn_hbm, dst_hbm, out_hbm, idx_v, rows_v, acc_sh):
        cid = lax.axis_index("c")
        sid = lax.axis_index("s")
        col0 = cid * _i32(DH)

        def zr(r, _):
            def zc(k, _):
                rows_v[r, pl.ds(k * L, L)] = jnp.zeros((L,), jnp.float32)
                return 0
            lax.fori_loop(_i32(0), _i32(DH // L), zc, 0)
            return 0

        lax.fori_loop(_i32(0), _i32(R_CH), zr, 0)

        def zs(b, _):
            pltpu.sync_copy(
                rows_v, acc_sh.at[pl.ds(sid * _i32(STRIPE) + b * _i32(R_CH), R_CH)])
            return 0

        lax.fori_loop(_i32(0), _i32(STRIPE // R_CH), zs, 0)
        plsc.subcore_barrier()

        def body(c, _):
            chid = c * _i32(NS) + sid

            @pl.when(chid < _i32(R_NCH))
            def _():
                off = chid * _i32(R_CH)
                pltpu.sync_copy(dst_hbm.at[pl.ds(off, R_CH)], idx_v)
                pltpu.sync_copy(
                    attn_hbm.at[pl.ds(off, R_CH), pl.ds(col0, DH)], rows_v)
                pltpu.sync_copy(rows_v, acc_sh.at[idx_v], add=True)
            return 0

        lax.fori_loop(_i32(0), _i32(R_IT), body, 0)
        plsc.subcore_barrier()

        pltpu.sync_copy(
            acc_sh.at[pl.ds(sid * _i32(STRIPE), STRIPE)],
            out_hbm.at[pl.ds(sid * _i32(STRIPE), STRIPE), pl.ds(col0, DH)])

    return rk(attn, dst)


def kernel(feat, edge_index, W):
    feat = feat.astype(jnp.float32)
    W = W.astype(jnp.float32)
    src = edge_index[0].astype(jnp.int32)
    dst = edge_index[1].astype(jnp.int32)

    ft = _matmul(feat, W)
    fs = _sc_gather(ft, src)
    fd = _sc_gather(ft, dst)
    fs3 = fs.reshape(E_ROWS, E_BLK, D)
    fd3 = fd.reshape(E_ROWS, E_BLK, D)
    e = _edge_dots(fs3, fd3)
    emax, den = _sc_maxdenom(e, dst)
    w = _sc_weights(e, dst, emax, den)
    attn3 = _attn_scale(fs3, w.reshape(E_ROWS, 1, E_BLK))
    out = _sc_scatter_rows(attn3.reshape(N_E, D), dst)
    return out[:N_N]


# speedup vs baseline: 1.4462x; 1.1789x over previous
import functools

import jax
import jax.numpy as jnp
from jax import lax
from jax.experimental import pallas as pl
from jax.experimental.pallas import tpu as pltpu
from jax.experimental.pallas import tpu_sc as plsc

N_N = 10000
N_E = 160000
D = 256

NC = 2
NS = 16
NW = NC * NS
L = 16

NP = 10240
NODES_W = NP // NW

_i32 = jnp.int32


def _mesh():
    return plsc.VectorSubcoreMesh(core_axis_name="c", subcore_axis_name="s")


def _wid():
    return lax.axis_index("s") * NC + lax.axis_index("c")


def _mm_body(x_ref, w_ref, o_ref):
    o_ref[...] = jnp.dot(x_ref[...], w_ref[...],
                         preferred_element_type=jnp.float32)


def _matmul(feat, W):
    return pl.pallas_call(
        _mm_body,
        out_shape=jax.ShapeDtypeStruct((N_N, D), jnp.float32),
    )(feat, W)


G_CH = 128
G_NCH = N_E // G_CH
G_IT = (G_NCH + NW - 1) // NW


def _sc_gather(table, idx):
    @functools.partial(
        pl.kernel, mesh=_mesh(),
        compiler_params=pltpu.CompilerParams(needs_layout_passes=False),
        out_type=jax.ShapeDtypeStruct((N_E, D), jnp.float32),
        scratch_types=[
            pltpu.VMEM((G_CH,), jnp.int32),
            pltpu.VMEM((G_CH,), jnp.int32),
            pltpu.VMEM((G_CH, D), jnp.float32),
            pltpu.VMEM((G_CH, D), jnp.float32),
            pltpu.SemaphoreType.DMA,
            pltpu.SemaphoreType.DMA,
        ],
    )
    def gk(table_hbm, idx_hbm, out_hbm, idx_a, idx_b, rows_a, rows_b,
           sem_a, sem_b):
        w = _wid()

        def body(c, _):
            cid_a = (c * _i32(2)) * _i32(NW) + w
            cid_b = (c * _i32(2) + _i32(1)) * _i32(NW) + w

            @pl.when(cid_a < _i32(G_NCH))
            def _():
                off = cid_a * _i32(G_CH)
                pltpu.sync_copy(idx_hbm.at[pl.ds(off, G_CH)], idx_a)
                pltpu.async_copy(table_hbm.at[idx_a], rows_a, sem_a)

            @pl.when(cid_b < _i32(G_NCH))
            def _():
                off = cid_b * _i32(G_CH)
                pltpu.sync_copy(idx_hbm.at[pl.ds(off, G_CH)], idx_b)
                pltpu.async_copy(table_hbm.at[idx_b], rows_b, sem_b)

            @pl.when(cid_a < _i32(G_NCH))
            def _():
                off = cid_a * _i32(G_CH)
                pltpu.make_async_copy(table_hbm.at[idx_a], rows_a,
                                      sem_a).wait()
                pltpu.sync_copy(rows_a, out_hbm.at[pl.ds(off, G_CH)])

            @pl.when(cid_b < _i32(G_NCH))
            def _():
                off = cid_b * _i32(G_CH)
                pltpu.make_async_copy(table_hbm.at[idx_b], rows_b,
                                      sem_b).wait()
                pltpu.sync_copy(rows_b, out_hbm.at[pl.ds(off, G_CH)])
            return 0

        lax.fori_loop(_i32(0), _i32((G_IT + 1) // 2), body, 0)

    return gk(table, idx)


E_BLK = 128
E_ROWS = N_E // E_BLK


def _dot_body(a_ref, b_ref, o_ref):
    o_ref[...] = jnp.sum(a_ref[...] * b_ref[...], axis=-1,
                         keepdims=True).reshape(1, 1, E_BLK)


def _edge_dots(fs3, fd3):
    z = _i32(0)
    e3 = pl.pallas_call(
        _dot_body,
        grid=(E_ROWS,),
        in_specs=[
            pl.BlockSpec((1, E_BLK, D), lambda i: (i, _i32(0), _i32(0))),
            pl.BlockSpec((1, E_BLK, D), lambda i: (i, _i32(0), _i32(0))),
        ],
        out_specs=pl.BlockSpec((1, 1, E_BLK), lambda i: (i, _i32(0), _i32(0))),
        out_shape=jax.ShapeDtypeStruct((E_ROWS, 1, E_BLK), jnp.float32),
    )(fs3, fd3)
    return e3.reshape(N_E)


S_CH = 2000
S_NCH = N_E // S_CH
NEG = -3.0e38


def _sc_maxdenom(e, dst):
    @functools.partial(
        pl.kernel, mesh=_mesh(),
        compiler_params=pltpu.CompilerParams(needs_layout_passes=False),
        out_type=(jax.ShapeDtypeStruct((NP,), jnp.float32),
                  jax.ShapeDtypeStruct((NP,), jnp.float32)),
        scratch_types=[
            pltpu.VMEM((S_CH,), jnp.float32),
            pltpu.VMEM((S_CH,), jnp.int32),
            pltpu.VMEM((NODES_W, L), jnp.float32),
            pltpu.VMEM((NODES_W, L), jnp.float32),
            pltpu.VMEM((NODES_W,), jnp.float32),
            pltpu.VMEM((NODES_W,), jnp.float32),
        ],
    )
    def mk(e_hbm, dst_hbm, emax_hbm, den_hbm,
           e_v, d_v, mx_t, sm_t, mx_r, sm_r):
        w = _wid()
        lo = w * _i32(NODES_W)
        lane = lax.iota(jnp.int32, L)

        def init_row(r, _):
            mx_t[r, :] = jnp.full((L,), NEG, jnp.float32)
            sm_t[r, :] = jnp.zeros((L,), jnp.float32)
            return 0

        lax.fori_loop(_i32(0), _i32(NODES_W), init_row, 0)

        def load_chunk(c):
            off = c * _i32(S_CH)
            pltpu.sync_copy(e_hbm.at[pl.ds(off, S_CH)], e_v)
            pltpu.sync_copy(dst_hbm.at[pl.ds(off, S_CH)], d_v)

        def scan1(c, _):
            load_chunk(c)

            def grp(j, _):
                ev = e_v[pl.ds(j * _i32(L), L)]
                dv = d_v[pl.ds(j * _i32(L), L)]
                m = (dv >= lo) & (dv < lo + _i32(NODES_W))
                loc = jnp.where(m, dv - lo, 0)
                cur = plsc.load_gather(mx_t, [loc, lane], mask=m)
                plsc.store_scatter(mx_t, [loc, lane],
                                   jnp.maximum(cur, ev), mask=m)
                return 0

            lax.fori_loop(_i32(0), _i32(S_CH // L), grp, 0)
            return 0

        lax.fori_loop(_i32(0), _i32(S_NCH), scan1, 0)

        def redmax(rg, _):
            rows = lax.iota(jnp.int32, L) + rg * _i32(L)
            acc = jnp.full((L,), NEG, jnp.float32)

            def col(c, a):
                cc = jnp.full((L,), 0, jnp.int32) + c
                return jnp.maximum(a, plsc.load_gather(mx_t, [rows, cc]))

            acc = lax.fori_loop(_i32(0), _i32(L), col, acc)
            mx_r[pl.ds(rg * _i32(L), L)] = acc
            return 0

        lax.fori_loop(_i32(0), _i32(NODES_W // L), redmax, 0)

        def scan2(c, _):
            load_chunk(c)

            def grp(j, _):
                ev = e_v[pl.ds(j * _i32(L), L)]
                dv = d_v[pl.ds(j * _i32(L), L)]
                m = (dv >= lo) & (dv < lo + _i32(NODES_W))
                loc = jnp.where(m, dv - lo, 0)
                mv = plsc.load_gather(mx_r, [loc], mask=m)
                ex = jnp.where(m, jnp.exp(ev - mv), jnp.float32(0.0))
                plsc.addupdate_scatter(sm_t, [loc, lane], ex, mask=m)
                return 0

            lax.fori_loop(_i32(0), _i32(S_CH // L), grp, 0)
            return 0

        lax.fori_loop(_i32(0), _i32(S_NCH), scan2, 0)

        def redsum(rg, _):
            rows = lax.iota(jnp.int32, L) + rg * _i32(L)
            acc = jnp.zeros((L,), jnp.float32)

            def col(c, a):
                cc = jnp.full((L,), 0, jnp.int32) + c
                return a + plsc.load_gather(sm_t, [rows, cc])

            acc = lax.fori_loop(_i32(0), _i32(L), col, acc)
            sm_r[pl.ds(rg * _i32(L), L)] = acc
            return 0

        lax.fori_loop(_i32(0), _i32(NODES_W // L), redsum, 0)

        pltpu.sync_copy(mx_r, emax_hbm.at[pl.ds(lo, NODES_W)])
        pltpu.sync_copy(sm_r, den_hbm.at[pl.ds(lo, NODES_W)])

    return mk(e, dst)


W_CH = 128
W_NCH = N_E // W_CH
W_IT = (W_NCH + NW - 1) // NW


def _sc_weights(e, dst, emax, den):
    @functools.partial(
        pl.kernel, mesh=_mesh(),
        compiler_params=pltpu.CompilerParams(needs_layout_passes=False),
        out_type=jax.ShapeDtypeStruct((N_E,), jnp.float32),
        scratch_types=[
            pltpu.VMEM((W_CH,), jnp.float32),
            pltpu.VMEM((W_CH,), jnp.int32),
            pltpu.VMEM((W_CH,), jnp.float32),
            pltpu.VMEM((W_CH,), jnp.float32),
            pltpu.VMEM((W_CH,), jnp.float32),
            pltpu.SemaphoreType.DMA,
        ],
    )
    def wk(e_hbm, dst_hbm, emax_hbm, den_hbm, w_hbm,
           e_v, d_v, m_v, s_v, w_v, sem):
        w = _wid()

        def body(c, _):
            cid = c * _i32(NW) + w

            @pl.when(cid < _i32(W_NCH))
            def _():
                off = cid * _i32(W_CH)
                pltpu.sync_copy(e_hbm.at[pl.ds(off, W_CH)], e_v)
                pltpu.sync_copy(dst_hbm.at[pl.ds(off, W_CH)], d_v)
                pltpu.async_copy(emax_hbm.at[d_v], m_v, sem).wait()
                pltpu.async_copy(den_hbm.at[d_v], s_v, sem).wait()

                def grp(j, _):
                    sl = pl.ds(j * _i32(L), L)
                    w_v[sl] = jnp.exp(e_v[sl] - m_v[sl]) / s_v[sl]
                    return 0

                lax.fori_loop(_i32(0), _i32(W_CH // L), grp, 0)
                pltpu.sync_copy(w_v, w_hbm.at[pl.ds(off, W_CH)])
            return 0

        lax.fori_loop(_i32(0), _i32(W_IT), body, 0)

    return wk(e, dst, emax, den)


def _attn_body(a_ref, w_ref, o_ref):
    o_ref[...] = a_ref[...] * w_ref[...].reshape(1, E_BLK, 1)


def _attn_scale(fs3, w2):
    return pl.pallas_call(
        _attn_body,
        grid=(E_ROWS,),
        in_specs=[
            pl.BlockSpec((1, E_BLK, D), lambda i: (i, _i32(0), _i32(0))),
            pl.BlockSpec((1, 1, E_BLK), lambda i: (i, _i32(0), _i32(0))),
        ],
        out_specs=pl.BlockSpec((1, E_BLK, D), lambda i: (i, _i32(0), _i32(0))),
        out_shape=jax.ShapeDtypeStruct((E_ROWS, E_BLK, D), jnp.float32),
    )(fs3, w2)


DH = D // NC
R_CH = 128
R_NCH = N_E // R_CH
R_IT = (R_NCH + NS - 1) // NS
STRIPE = NP // NS


def _sc_scatter_rows(attn, dst):
    @functools.partial(
        pl.kernel, mesh=_mesh(),
        compiler_params=pltpu.CompilerParams(needs_layout_passes=False),
        out_type=jax.ShapeDtypeStruct((NP, D), jnp.float32),
        scratch_types=[
            pltpu.VMEM((R_CH,), jnp.int32),
            pltpu.VMEM((R_CH,), jnp.int32),
            pltpu.VMEM((R_CH, DH), jnp.float32),
            pltpu.VMEM((R_CH, DH), jnp.float32),
            pltpu.VMEM_SHARED((NP, DH), jnp.float32),
            pltpu.SemaphoreType.DMA,
            pltpu.SemaphoreType.DMA,
        ],
    )
    def rk(attn_hbm, dst_hbm, out_hbm, idx_v, idx_b, rows_v, rows_b,
           acc_sh, sem_a, sem_b):
        cid = lax.axis_index("c")
        sid = lax.axis_index("s")
        col0 = cid * _i32(DH)

        def zr(r, _):
            def zc(k, _):
                rows_v[r, pl.ds(k * L, L)] = jnp.zeros((L,), jnp.float32)
                return 0
            lax.fori_loop(_i32(0), _i32(DH // L), zc, 0)
            return 0

        lax.fori_loop(_i32(0), _i32(R_CH), zr, 0)

        def zs(b, _):
            pltpu.sync_copy(
                rows_v, acc_sh.at[pl.ds(sid * _i32(STRIPE) + b * _i32(R_CH), R_CH)])
            return 0

        lax.fori_loop(_i32(0), _i32(STRIPE // R_CH), zs, 0)
        plsc.subcore_barrier()

        def body(c, _):
            chid_a = (c * _i32(2)) * _i32(NS) + sid
            chid_b = (c * _i32(2) + _i32(1)) * _i32(NS) + sid

            @pl.when(chid_a < _i32(R_NCH))
            def _():
                off = chid_a * _i32(R_CH)
                pltpu.sync_copy(dst_hbm.at[pl.ds(off, R_CH)], idx_v)
                pltpu.async_copy(
                    attn_hbm.at[pl.ds(off, R_CH), pl.ds(col0, DH)],
                    rows_v, sem_a)

            @pl.when(chid_b < _i32(R_NCH))
            def _():
                off = chid_b * _i32(R_CH)
                pltpu.sync_copy(dst_hbm.at[pl.ds(off, R_CH)], idx_b)
                pltpu.async_copy(
                    attn_hbm.at[pl.ds(off, R_CH), pl.ds(col0, DH)],
                    rows_b, sem_b)

            @pl.when(chid_a < _i32(R_NCH))
            def _():
                off = chid_a * _i32(R_CH)
                pltpu.make_async_copy(
                    attn_hbm.at[pl.ds(off, R_CH), pl.ds(col0, DH)],
                    rows_v, sem_a).wait()
                pltpu.sync_copy(rows_v, acc_sh.at[idx_v], add=True)

            @pl.when(chid_b < _i32(R_NCH))
            def _():
                off = chid_b * _i32(R_CH)
                pltpu.make_async_copy(
                    attn_hbm.at[pl.ds(off, R_CH), pl.ds(col0, DH)],
                    rows_b, sem_b).wait()
                pltpu.sync_copy(rows_b, acc_sh.at[idx_b], add=True)
            return 0

        lax.fori_loop(_i32(0), _i32((R_IT + 1) // 2), body, 0)
        plsc.subcore_barrier()

        pltpu.sync_copy(
            acc_sh.at[pl.ds(sid * _i32(STRIPE), STRIPE)],
            out_hbm.at[pl.ds(sid * _i32(STRIPE), STRIPE), pl.ds(col0, DH)])

    return rk(attn, dst)


def kernel(feat, edge_index, W):
    feat = feat.astype(jnp.float32)
    W = W.astype(jnp.float32)
    src = edge_index[0].astype(jnp.int32)
    dst = edge_index[1].astype(jnp.int32)

    ft = _matmul(feat, W)
    fs = _sc_gather(ft, src)
    fd = _sc_gather(ft, dst)
    fs3 = fs.reshape(E_ROWS, E_BLK, D)
    fd3 = fd.reshape(E_ROWS, E_BLK, D)
    e = _edge_dots(fs3, fd3)
    emax, den = _sc_maxdenom(e, dst)
    w = _sc_weights(e, dst, emax, den)
    attn3 = _attn_scale(fs3, w.reshape(E_ROWS, 1, E_BLK))
    out = _sc_scatter_rows(attn3.reshape(N_E, D), dst)
    return out[:N_N]
